# Initial kernel scaffold; baseline (speedup 1.0000x reference)
import jax
import jax.numpy as jnp
from jax.experimental import pallas as pl


def kernel(score, s_e_time):
    raise NotImplementedError("write your pallas kernel here")


# speedup vs baseline: 11.3977x; 11.3977x over previous
import jax
import jax.numpy as jnp
from jax.experimental import pallas as pl

_NMS_THRESHOLD = 0.5
_NUM_CLIPS = 64
_K = 2048


def _nms_kernel(sc_ref, t0_ref, t1_ref, b_ref, box_ref, score_ref, keep_ref):
    t0 = t0_ref[...]
    t1 = t1_ref[...]
    sraw = jnp.minimum(t0, t1)
    eraw = jnp.maximum(t0, t1)
    s = sraw * float(_NUM_CLIPS)
    e = eraw * float(_NUM_CLIPS)
    length = e - s
    b = b_ref[...]
    sc = sc_ref[...]
    iota = jax.lax.broadcasted_iota(jnp.int32, (1, _K), 1)
    valid = (sc > -1e8).astype(jnp.float32)

    def body(i, keep):
        onehot = iota == i
        ki = jnp.sum(jnp.where(onehot, keep, 0.0))
        si = jnp.sum(jnp.where(onehot, s, 0.0))
        ei = jnp.sum(jnp.where(onehot, e, 0.0))
        bi = jnp.sum(jnp.where(onehot, b, 0))
        inter = jnp.maximum(jnp.minimum(ei, e) - jnp.maximum(si, s), 0.0)
        union = jnp.maximum((ei - si) + length - inter, 1e-6)
        sup = (b == bi) & (inter > _NMS_THRESHOLD * union) & (iota > i)
        return jnp.where(sup & (ki > 0.0), 0.0, keep)

    keep = jax.lax.fori_loop(0, _K, body, valid)
    score_ref[...] = sc * keep
    keep_ref[...] = keep
    box_ref[0:1, :] = sraw * keep
    box_ref[1:2, :] = eraw * keep


def kernel(score, s_e_time):
    B, ST, ED = score.shape
    N = ST * ED
    scores = score.reshape(B * N)
    scores_m = jnp.where(scores > 0, scores, -1e9)
    t = s_e_time.reshape(B, 2, N)
    t0 = t[:, 0, :].reshape(B * N)
    t1 = t[:, 1, :].reshape(B * N)
    top_scores, top_idx = jax.lax.top_k(scores_m, _K)
    g0 = jnp.take(t0, top_idx)
    g1 = jnp.take(t1, top_idx)
    gb = (top_idx // N).astype(jnp.int32)

    box_t, nms_s, keep_f = pl.pallas_call(
        _nms_kernel,
        out_shape=(
            jax.ShapeDtypeStruct((2, _K), jnp.float32),
            jax.ShapeDtypeStruct((1, _K), jnp.float32),
            jax.ShapeDtypeStruct((1, _K), jnp.float32),
        ),
    )(
        top_scores.reshape(1, _K),
        g0.reshape(1, _K),
        g1.reshape(1, _K),
        gb.reshape(1, _K),
    )
    boxxes = box_t.T
    nms_score = nms_s.reshape(_K)
    keep = keep_f.reshape(_K) != 0
    return boxxes, nms_score, keep


# speedup vs baseline: 14.0353x; 1.2314x over previous
import jax
import jax.numpy as jnp
from jax.experimental import pallas as pl

_NMS_THRESHOLD = 0.5
_NUM_CLIPS = 64
_K = 2048
_W = 128
_C = _K // _W


def _nms_kernel(sc_ref, t0_ref, t1_ref, b_ref, box_ref, score_ref, keep_ref):
    t0 = t0_ref[...]
    t1 = t1_ref[...]
    sraw = jnp.minimum(t0, t1)
    eraw = jnp.maximum(t0, t1)
    s = sraw * float(_NUM_CLIPS)
    e = eraw * float(_NUM_CLIPS)
    length = e - s
    b = b_ref[...]
    sc = sc_ref[...]
    iota_w = jax.lax.broadcasted_iota(jnp.int32, (1, _W), 1)
    valid = (sc > -1e8).astype(jnp.float32)

    keeps = [valid[:, c * _W:(c + 1) * _W] for c in range(_C)]
    for c in range(_C):
        base = c * _W
        s_c = s[:, base:base + _W]
        e_c = e[:, base:base + _W]
        l_c = length[:, base:base + _W]
        b_c = b[:, base:base + _W]

        def body(i, kc, s_c=s_c, e_c=e_c, l_c=l_c, b_c=b_c):
            onehot = iota_w == i
            ki = jnp.sum(jnp.where(onehot, kc, 0.0))
            si = jnp.sum(jnp.where(onehot, s_c, 0.0))
            ei = jnp.sum(jnp.where(onehot, e_c, 0.0))
            bi = jnp.sum(jnp.where(onehot, b_c, 0))
            inter = jnp.maximum(jnp.minimum(ei, e_c) - jnp.maximum(si, s_c), 0.0)
            union = jnp.maximum((ei - si) + l_c - inter, 1e-6)
            sup = (b_c == bi) & (inter > _NMS_THRESHOLD * union) & (iota_w > i)
            return jnp.where(sup & (ki > 0.0), 0.0, kc)

        kc = jax.lax.fori_loop(0, _W, body, keeps[c])
        keeps[c] = kc

        if c + 1 < _C:
            lo = base + _W
            sp = s_c.reshape(_W, 1)
            ep = e_c.reshape(_W, 1)
            lp = l_c.reshape(_W, 1)
            bp = b_c.reshape(_W, 1)
            kp = kc.reshape(_W, 1)
            s_r = s[:, lo:]
            e_r = e[:, lo:]
            l_r = length[:, lo:]
            b_r = b[:, lo:]
            inter = jnp.maximum(jnp.minimum(ep, e_r) - jnp.maximum(sp, s_r), 0.0)
            union = jnp.maximum(lp + l_r - inter, 1e-6)
            supm = (bp == b_r) & (inter > _NMS_THRESHOLD * union) & (kp > 0.0)
            any_sup = jnp.any(supm, axis=0, keepdims=True)
            for cc in range(c + 1, _C):
                off = (cc - c - 1) * _W
                keeps[cc] = jnp.where(any_sup[:, off:off + _W], 0.0, keeps[cc])

    keep = jnp.concatenate(keeps, axis=1)
    score_ref[...] = sc * keep
    keep_ref[...] = keep
    box_ref[0:1, :] = sraw * keep
    box_ref[1:2, :] = eraw * keep


def kernel(score, s_e_time):
    B, ST, ED = score.shape
    N = ST * ED
    scores = score.reshape(B * N)
    scores_m = jnp.where(scores > 0, scores, -1e9)
    t = s_e_time.reshape(B, 2, N)
    t0 = t[:, 0, :].reshape(B * N)
    t1 = t[:, 1, :].reshape(B * N)
    top_scores, top_idx = jax.lax.top_k(scores_m, _K)
    g0 = jnp.take(t0, top_idx)
    g1 = jnp.take(t1, top_idx)
    gb = (top_idx // N).astype(jnp.int32)

    box_t, nms_s, keep_f = pl.pallas_call(
        _nms_kernel,
        out_shape=(
            jax.ShapeDtypeStruct((2, _K), jnp.float32),
            jax.ShapeDtypeStruct((1, _K), jnp.float32),
            jax.ShapeDtypeStruct((1, _K), jnp.float32),
        ),
    )(
        top_scores.reshape(1, _K),
        g0.reshape(1, _K),
        g1.reshape(1, _K),
        gb.reshape(1, _K),
    )
    boxxes = box_t.T
    nms_score = nms_s.reshape(_K)
    keep = keep_f.reshape(_K) != 0
    return boxxes, nms_score, keep


# speedup vs baseline: 22.1672x; 1.5794x over previous
import jax
import jax.numpy as jnp
from jax.experimental import pallas as pl

_NMS_THRESHOLD = 0.5
_NUM_CLIPS = 64
_K = 2048
_W = 128
_C = _K // _W


def _nms_kernel(sc_ref, t0_ref, t1_ref, b_ref, box_ref, score_ref, keep_ref):
    t0 = t0_ref[...]
    t1 = t1_ref[...]
    sraw = jnp.minimum(t0, t1)
    eraw = jnp.maximum(t0, t1)
    s = sraw * float(_NUM_CLIPS)
    e = eraw * float(_NUM_CLIPS)
    length = e - s
    b = b_ref[...]
    sc = sc_ref[...]
    iota_w = jax.lax.broadcasted_iota(jnp.int32, (1, _W), 1)
    valid = (sc > -1e8).astype(jnp.float32)

    keeps = [valid[:, c * _W:(c + 1) * _W] for c in range(_C)]
    for c in range(_C):
        base = c * _W
        s_c = s[:, base:base + _W]
        e_c = e[:, base:base + _W]
        l_c = length[:, base:base + _W]
        b_c = b[:, base:base + _W]

        kc = keeps[c]
        for i in range(_W - 1):
            ki = kc[:, i:i + 1]
            si = s_c[:, i:i + 1]
            ei = e_c[:, i:i + 1]
            bi = b_c[:, i:i + 1]
            inter = jnp.maximum(jnp.minimum(ei, e_c) - jnp.maximum(si, s_c), 0.0)
            union = jnp.maximum((ei - si) + l_c - inter, 1e-6)
            sup = (b_c == bi) & (inter > _NMS_THRESHOLD * union) & (iota_w > i)
            kc = jnp.where(sup & (ki > 0.0), 0.0, kc)
        keeps[c] = kc

        if c + 1 < _C:
            lo = base + _W
            sp = s_c.reshape(_W, 1)
            ep = e_c.reshape(_W, 1)
            lp = l_c.reshape(_W, 1)
            bp = b_c.reshape(_W, 1)
            kp = kc.reshape(_W, 1)
            s_r = s[:, lo:]
            e_r = e[:, lo:]
            l_r = length[:, lo:]
            b_r = b[:, lo:]
            inter = jnp.maximum(jnp.minimum(ep, e_r) - jnp.maximum(sp, s_r), 0.0)
            union = jnp.maximum(lp + l_r - inter, 1e-6)
            supm = (bp == b_r) & (inter > _NMS_THRESHOLD * union) & (kp > 0.0)
            any_sup = jnp.any(supm, axis=0, keepdims=True)
            for cc in range(c + 1, _C):
                off = (cc - c - 1) * _W
                keeps[cc] = jnp.where(any_sup[:, off:off + _W], 0.0, keeps[cc])

    keep = jnp.concatenate(keeps, axis=1)
    score_ref[...] = sc * keep
    keep_ref[...] = keep
    box_ref[0:1, :] = sraw * keep
    box_ref[1:2, :] = eraw * keep


def kernel(score, s_e_time):
    B, ST, ED = score.shape
    N = ST * ED
    scores = score.reshape(B * N)
    scores_m = jnp.where(scores > 0, scores, -1e9)
    t = s_e_time.reshape(B, 2, N)
    t0 = t[:, 0, :].reshape(B * N)
    t1 = t[:, 1, :].reshape(B * N)
    top_scores, top_idx = jax.lax.top_k(scores_m, _K)
    g0 = jnp.take(t0, top_idx)
    g1 = jnp.take(t1, top_idx)
    gb = (top_idx // N).astype(jnp.int32)

    box_t, nms_s, keep_f = pl.pallas_call(
        _nms_kernel,
        out_shape=(
            jax.ShapeDtypeStruct((2, _K), jnp.float32),
            jax.ShapeDtypeStruct((1, _K), jnp.float32),
            jax.ShapeDtypeStruct((1, _K), jnp.float32),
        ),
    )(
        top_scores.reshape(1, _K),
        g0.reshape(1, _K),
        g1.reshape(1, _K),
        gb.reshape(1, _K),
    )
    boxxes = box_t.T
    nms_score = nms_s.reshape(_K)
    keep = keep_f.reshape(_K) != 0
    return boxxes, nms_score, keep
